# speedup vs baseline: 1.6581x; 1.6581x over previous
import functools

import jax
import jax.numpy as jnp
from jax import lax
from jax.experimental import pallas as pl
from jax.experimental.pallas import tpu as pltpu
from jax.experimental.pallas import tpu_sc as plsc

H = 128
L = 16
NC = 2
NS = 16
NW = NC * NS
ROWS = 8192
RPW = ROWS // NW
SEQ = 2048

_mesh = plsc.VectorSubcoreMesh(core_axis_name="c", subcore_axis_name="s")


@functools.partial(
    pl.kernel,
    out_type=jax.ShapeDtypeStruct((ROWS, H), jnp.float32),
    mesh=_mesh,
    scratch_types=[
        pltpu.VMEM((RPW,), jnp.int32),
        pltpu.VMEM((RPW, H), jnp.float32),
        pltpu.VMEM((RPW, H), jnp.float32),
        pltpu.SemaphoreType.DMA,
    ],
)
def _embed_lookup(idx_hbm, tok_hbm, pos_hbm, out_hbm, idx_v, tok_v, pos_v, sem):
    wid = lax.axis_index("s") * NC + lax.axis_index("c")
    base = wid * RPW
    s_base = lax.rem(base, SEQ)

    pltpu.sync_copy(idx_hbm.at[pl.ds(base, RPW)], idx_v)
    gather = pltpu.async_copy(tok_hbm.at[idx_v], tok_v, sem)
    pltpu.sync_copy(pos_hbm.at[pl.ds(s_base, RPW), :], pos_v)
    gather.wait()

    def body(j, carry):
        for c in range(H // L):
            sl = pl.ds(c * L, L)
            tok_v[j, sl] = tok_v[j, sl] + pos_v[j, sl]
        return carry

    lax.fori_loop(0, RPW, body, 0)

    pltpu.sync_copy(tok_v, out_hbm.at[pl.ds(base, RPW)])


def kernel(x, token_table, position_table):
    idx = x.reshape(-1).astype(jnp.int32)
    out = _embed_lookup(idx, token_table, position_table)
    return out.reshape(x.shape + (H,))


# speedup vs baseline: 1.7050x; 1.0283x over previous
import functools

import jax
import jax.numpy as jnp
from jax import lax
from jax.experimental import pallas as pl
from jax.experimental.pallas import tpu as pltpu
from jax.experimental.pallas import tpu_sc as plsc

H = 128
L = 16
NC = 2
NS = 16
NW = NC * NS
BATCH = 4
SEQ = 2048
ROWS = BATCH * SEQ
RPW = ROWS // NW
WPB = SEQ // RPW
CHUNK = RPW // 2

_mesh = plsc.VectorSubcoreMesh(core_axis_name="c", subcore_axis_name="s")


@functools.partial(
    pl.kernel,
    out_type=jax.ShapeDtypeStruct((BATCH, SEQ, H), jnp.float32),
    mesh=_mesh,
    scratch_types=[
        pltpu.VMEM((RPW,), jnp.int32),
        pltpu.VMEM((CHUNK, H), jnp.float32),
        pltpu.VMEM((CHUNK, H), jnp.float32),
        pltpu.VMEM((RPW, H), jnp.float32),
        pltpu.SemaphoreType.DMA,
        pltpu.SemaphoreType.DMA,
        pltpu.SemaphoreType.DMA,
    ],
)
def _embed_lookup(x_hbm, tok_hbm, pos_hbm, out_hbm,
                  idx_v, tok0_v, tok1_v, pos_v, g0_sem, g1_sem, out_sem):
    wid = lax.axis_index("s") * NC + lax.axis_index("c")
    b = wid // WPB
    s0 = (wid % WPB) * RPW

    pltpu.sync_copy(x_hbm.at[b, pl.ds(s0, RPW)], idx_v)
    g0 = pltpu.async_copy(tok_hbm.at[idx_v.at[pl.ds(0, CHUNK)]], tok0_v, g0_sem)
    g1 = pltpu.async_copy(tok_hbm.at[idx_v.at[pl.ds(CHUNK, CHUNK)]], tok1_v, g1_sem)
    pltpu.sync_copy(pos_hbm.at[pl.ds(s0, RPW), :], pos_v)

    def add_rows(tok_ref, pos_off):
        def body(j, carry):
            for c in range(H // L):
                sl = pl.ds(c * L, L)
                plsc.addupdate(tok_ref.at[j, sl], pos_v[pos_off + j, sl])
            return carry
        lax.fori_loop(0, CHUNK, body, 0)

    g0.wait()
    add_rows(tok0_v, 0)
    o0 = pltpu.async_copy(tok0_v, out_hbm.at[b, pl.ds(s0, CHUNK), :], out_sem)

    g1.wait()
    add_rows(tok1_v, CHUNK)
    o0.wait()
    pltpu.sync_copy(tok1_v, out_hbm.at[b, pl.ds(s0 + CHUNK, CHUNK), :])


def kernel(x, token_table, position_table):
    return _embed_lookup(x.astype(jnp.int32), token_table, position_table)


# speedup vs baseline: 1.7390x; 1.0199x over previous
import functools

import jax
import jax.numpy as jnp
from jax import lax
from jax.experimental import pallas as pl
from jax.experimental.pallas import tpu as pltpu
from jax.experimental.pallas import tpu_sc as plsc

H = 128
L = 16
NC = 2
NS = 16
NW = NC * NS
BATCH = 4
SEQ = 2048
CHUNK = 128
NSLICE = NW // (BATCH // 2)

_mesh = plsc.VectorSubcoreMesh(core_axis_name="c", subcore_axis_name="s")


@functools.partial(
    pl.kernel,
    out_type=jax.ShapeDtypeStruct((BATCH, SEQ, H), jnp.float32),
    mesh=_mesh,
    scratch_types=[
        pltpu.VMEM((CHUNK,), jnp.int32),
        pltpu.VMEM((CHUNK,), jnp.int32),
        pltpu.VMEM((CHUNK, H), jnp.float32),
        pltpu.VMEM((CHUNK, H), jnp.float32),
        pltpu.VMEM((CHUNK, H), jnp.float32),
        pltpu.SemaphoreType.DMA,
        pltpu.SemaphoreType.DMA,
        pltpu.SemaphoreType.DMA,
    ],
)
def _embed_lookup(x_hbm, tok_hbm, pos_hbm, out_hbm,
                  idxa_v, idxb_v, toka_v, tokb_v, pos_v,
                  ga_sem, gb_sem, out_sem):
    wid = lax.axis_index("s") * NC + lax.axis_index("c")
    b0 = (wid // NSLICE) * 2
    s1 = (wid % NSLICE) * CHUNK

    pltpu.sync_copy(x_hbm.at[b0, pl.ds(s1, CHUNK)], idxa_v)
    ga = pltpu.async_copy(tok_hbm.at[idxa_v], toka_v, ga_sem)
    pltpu.sync_copy(x_hbm.at[b0 + 1, pl.ds(s1, CHUNK)], idxb_v)
    gb = pltpu.async_copy(tok_hbm.at[idxb_v], tokb_v, gb_sem)
    pltpu.sync_copy(pos_hbm.at[pl.ds(s1, CHUNK), :], pos_v)

    def add_rows(tok_ref):
        def body(j, carry):
            for c in range(H // L):
                sl = pl.ds(c * L, L)
                plsc.addupdate(tok_ref.at[j, sl], pos_v[j, sl])
            return carry
        lax.fori_loop(0, CHUNK, body, 0, unroll=2)

    ga.wait()
    add_rows(toka_v)
    oa = pltpu.async_copy(toka_v, out_hbm.at[b0, pl.ds(s1, CHUNK), :], out_sem)

    gb.wait()
    add_rows(tokb_v)
    oa.wait()
    pltpu.sync_copy(tokb_v, out_hbm.at[b0 + 1, pl.ds(s1, CHUNK), :])


def kernel(x, token_table, position_table):
    return _embed_lookup(x.astype(jnp.int32), token_table, position_table)


# speedup vs baseline: 1.7392x; 1.0001x over previous
import functools

import jax
import jax.numpy as jnp
from jax import lax
from jax.experimental import pallas as pl
from jax.experimental.pallas import tpu as pltpu
from jax.experimental.pallas import tpu_sc as plsc

H = 128
L = 16
NC = 2
NS = 16
NW = NC * NS
BATCH = 4
SEQ = 2048
PSLICE = 128
CHUNK = 64
NSLICE = SEQ // PSLICE

_mesh = plsc.VectorSubcoreMesh(core_axis_name="c", subcore_axis_name="s")


@functools.partial(
    pl.kernel,
    out_type=jax.ShapeDtypeStruct((BATCH, SEQ, H), jnp.float32),
    mesh=_mesh,
    scratch_types=[
        pltpu.VMEM((PSLICE,), jnp.int32),
        pltpu.VMEM((PSLICE,), jnp.int32),
        pltpu.VMEM((PSLICE, H), jnp.float32),
        [pltpu.VMEM((CHUNK, H), jnp.float32) for _ in range(4)],
        [pltpu.SemaphoreType.DMA for _ in range(4)],
        pltpu.SemaphoreType.DMA,
        pltpu.SemaphoreType.DMA,
    ],
)
def _embed_lookup(x_hbm, tok_hbm, pos_hbm, out_hbm,
                  idxa_v, idxb_v, pos_v, tok_bufs, g_sems, idx_sem, out_sem):
    wid = lax.axis_index("s") * NC + lax.axis_index("c")
    b0 = (wid // NSLICE) * 2
    s1 = (wid % NSLICE) * PSLICE

    ia = pltpu.async_copy(x_hbm.at[b0, pl.ds(s1, PSLICE)], idxa_v, idx_sem)
    ib = pltpu.async_copy(x_hbm.at[b0 + 1, pl.ds(s1, PSLICE)], idxb_v, idx_sem)
    ia.wait()
    g = [None] * 4
    g[0] = pltpu.async_copy(tok_hbm.at[idxa_v.at[pl.ds(0, CHUNK)]], tok_bufs[0], g_sems[0])
    g[1] = pltpu.async_copy(tok_hbm.at[idxa_v.at[pl.ds(CHUNK, CHUNK)]], tok_bufs[1], g_sems[1])
    ib.wait()
    g[2] = pltpu.async_copy(tok_hbm.at[idxb_v.at[pl.ds(0, CHUNK)]], tok_bufs[2], g_sems[2])
    g[3] = pltpu.async_copy(tok_hbm.at[idxb_v.at[pl.ds(CHUNK, CHUNK)]], tok_bufs[3], g_sems[3])
    pltpu.sync_copy(pos_hbm.at[pl.ds(s1, PSLICE), :], pos_v)

    def add_rows(tok_ref, pos_off):
        def body(j, carry):
            for c in range(H // L):
                sl = pl.ds(c * L, L)
                plsc.addupdate(tok_ref.at[j, sl], pos_v[pos_off + j, sl])
            return carry
        lax.fori_loop(0, CHUNK, body, 0, unroll=2)

    outs = []
    for k in range(4):
        g[k].wait()
        add_rows(tok_bufs[k], (k % 2) * CHUNK)
        dst = out_hbm.at[b0 + k // 2, pl.ds(s1 + (k % 2) * CHUNK, CHUNK), :]
        outs.append(pltpu.async_copy(tok_bufs[k], dst, out_sem))
    for o in outs:
        o.wait()


def kernel(x, token_table, position_table):
    return _embed_lookup(x.astype(jnp.int32), token_table, position_table)
